# baseline (speedup 1.0000x reference)
import jax
import jax.numpy as jnp
from jax import lax
from jax.experimental import pallas as pl
from jax.experimental.pallas import tpu as pltpu
from jax.experimental.pallas import tpu_sc as plsc

NC, NS, L = 2, 16, 16
NW = NC * NS
B = 4096
NT = 26
D = 128
NV = D // L
BPW = B // NW
BC = 8
NCHUNK = BPW // BC
ROWS = NT * BC
NBUF = 4
NIDX = NT * BPW
PER = 208

_mesh = plsc.VectorSubcoreMesh(core_axis_name="c", subcore_axis_name="s")

_scratch = [
    pltpu.VMEM((BPW, NT), jnp.int32),
    pltpu.VMEM((NIDX,), jnp.int32),
    pltpu.VMEM((ROWS, D), jnp.float32),
    pltpu.VMEM((ROWS, D), jnp.float32),
    pltpu.VMEM((ROWS, D), jnp.float32),
    pltpu.VMEM((ROWS, D), jnp.float32),
    pltpu.VMEM((BC, D), jnp.float32),
    pltpu.VMEM((BC, D), jnp.float32),
    pltpu.SemaphoreType.DMA,
    pltpu.SemaphoreType.DMA,
    pltpu.SemaphoreType.DMA,
    pltpu.SemaphoreType.DMA,
    pltpu.SemaphoreType.DMA,
]


def _embed_pool_body(act_hbm, table_hbm, out_hbm,
                     act_v, idx_v, gbuf0, gbuf1, gbuf2, gbuf3,
                     obuf0, obuf1, sem0, sem1, sem2, sem3, osem):
    wid = lax.axis_index("s") * NC + lax.axis_index("c")
    base = wid * BPW

    pltpu.sync_copy(act_hbm.at[pl.ds(base, BPW), :], act_v)

    lanes = lax.iota(jnp.int32, L)
    off_a = lanes * 1000
    off_b = off_a + 10000

    @pl.loop(0, BPW)
    def _mkidx(j):
        idx_v[pl.ds(j * NT, L)] = act_v[j, pl.ds(0, L)] + off_a
        idx_v[pl.ds(j * NT + NT - L, L)] = act_v[j, pl.ds(NT - L, L)] + off_b

    bufs = ((gbuf0, sem0), (gbuf1, sem1), (gbuf2, sem2), (gbuf3, sem3))

    def start_gather(c, buf, sem):
        pltpu.async_copy(table_hbm.at[idx_v.at[pl.ds(c * ROWS, ROWS)]], buf, sem)

    for b in range(NBUF):
        start_gather(b, *bufs[b])

    obufs = (obuf0, obuf1)

    @pl.loop(0, NCHUNK, step=NBUF)
    def _pair(c0):
        for b in range(NBUF):
            gbuf, sem = bufs[b]
            obuf = obufs[b % 2]
            c = c0 + b
            pltpu.make_async_copy(
                table_hbm.at[idx_v.at[pl.ds(c * ROWS, ROWS)]], gbuf, sem
            ).wait()

            @pl.when(c >= 2)
            def _():
                pltpu.make_async_copy(
                    obuf, out_hbm.at[pl.ds(base, BC)], osem
                ).wait()

            for jj in range(BC):
                def body(t, accs):
                    return tuple(
                        a + gbuf[jj * NT + t, pl.ds(v * L, L)]
                        for v, a in enumerate(accs)
                    )
                accs = tuple(gbuf[jj * NT, pl.ds(v * L, L)] for v in range(NV))
                accs = lax.fori_loop(1, NT, body, accs, unroll=5)
                for v in range(NV):
                    obuf[jj, pl.ds(v * L, L)] = accs[v]

            @pl.when(c + NBUF < NCHUNK)
            def _():
                start_gather(c + NBUF, gbuf, sem)

            pltpu.async_copy(obuf, out_hbm.at[pl.ds(base + c * BC, BC)], osem)

    for _ in range(2):
        pltpu.make_async_copy(obuf0, out_hbm.at[pl.ds(base, BC)], osem).wait()


_embed_pool = pl.kernel(
    _embed_pool_body,
    out_type=jax.ShapeDtypeStruct((B, D), jnp.float32),
    mesh=_mesh,
    scratch_types=_scratch,
)


def kernel(actions, embed_table):
    if actions.dtype != jnp.int32:
        actions = actions.astype(jnp.int32)
    return _embed_pool(actions, embed_table)


# speedup vs baseline: 1.0174x; 1.0174x over previous
import jax
import jax.numpy as jnp
from jax import lax
from jax.experimental import pallas as pl
from jax.experimental.pallas import tpu as pltpu
from jax.experimental.pallas import tpu_sc as plsc

NC, NS, L = 2, 16, 16
NW = NC * NS
B = 4096
NT = 26
D = 128
NV = D // L
BPW = B // NW
BC = 8
NCHUNK = BPW // BC
ROWS = NT * BC
NBUF = 4
NIDX = NT * BPW

_mesh = plsc.VectorSubcoreMesh(core_axis_name="c", subcore_axis_name="s")

_scratch = [
    pltpu.VMEM((BPW, NT), jnp.int32),
    pltpu.VMEM((NIDX,), jnp.int32),
    pltpu.VMEM((ROWS, D), jnp.float32),
    pltpu.VMEM((ROWS, D), jnp.float32),
    pltpu.VMEM((ROWS, D), jnp.float32),
    pltpu.VMEM((ROWS, D), jnp.float32),
    pltpu.VMEM((BC, D), jnp.float32),
    pltpu.VMEM((BC, D), jnp.float32),
    pltpu.SemaphoreType.DMA,
    pltpu.SemaphoreType.DMA,
    pltpu.SemaphoreType.DMA,
    pltpu.SemaphoreType.DMA,
    pltpu.SemaphoreType.DMA,
    pltpu.SemaphoreType.DMA,
]


def _embed_pool_body(act_hbm, table_hbm, out_hbm,
                     act_v, idx_v, gbuf0, gbuf1, gbuf2, gbuf3,
                     obuf0, obuf1, sem0, sem1, sem2, sem3, osem0, osem1):
    wid = lax.axis_index("s") * NC + lax.axis_index("c")
    base = wid * BPW

    pltpu.sync_copy(act_hbm.at[pl.ds(base, BPW), :], act_v)

    lanes = lax.iota(jnp.int32, L)
    off_a = lanes * 1000
    off_b = off_a + 10000

    @pl.loop(0, BPW)
    def _mkidx(j):
        idx_v[pl.ds(j * NT, L)] = act_v[j, pl.ds(0, L)] + off_a
        idx_v[pl.ds(j * NT + NT - L, L)] = act_v[j, pl.ds(NT - L, L)] + off_b

    bufs = ((gbuf0, sem0), (gbuf1, sem1), (gbuf2, sem2), (gbuf3, sem3))

    def start_gather(c, buf, sem):
        pltpu.async_copy(table_hbm.at[idx_v.at[pl.ds(c * ROWS, ROWS)]], buf, sem)

    for b in range(NBUF):
        start_gather(b, *bufs[b])

    obufs = ((obuf0, osem0), (obuf1, osem1))

    @pl.loop(0, NCHUNK, step=NBUF)
    def _pair(c0):
        for b in range(NBUF):
            gbuf, sem = bufs[b]
            obuf, osem = obufs[b % 2]
            c = c0 + b
            pltpu.make_async_copy(
                table_hbm.at[idx_v.at[pl.ds(c * ROWS, ROWS)]], gbuf, sem
            ).wait()

            @pl.when(c >= 2)
            def _():
                pltpu.make_async_copy(
                    obuf, out_hbm.at[pl.ds(base, BC)], osem
                ).wait()

            for jj in range(BC):
                def body(t, accs):
                    return tuple(
                        a + gbuf[jj * NT + t, pl.ds(v * L, L)]
                        for v, a in enumerate(accs)
                    )
                accs = tuple(gbuf[jj * NT, pl.ds(v * L, L)] for v in range(NV))
                accs = lax.fori_loop(1, NT, body, accs, unroll=5)
                for v in range(NV):
                    obuf[jj, pl.ds(v * L, L)] = accs[v]

            @pl.when(c + NBUF < NCHUNK)
            def _():
                start_gather(c + NBUF, gbuf, sem)

            pltpu.async_copy(obuf, out_hbm.at[pl.ds(base + c * BC, BC)], osem)

    pltpu.make_async_copy(obuf0, out_hbm.at[pl.ds(base, BC)], osem0).wait()
    pltpu.make_async_copy(obuf1, out_hbm.at[pl.ds(base, BC)], osem1).wait()


_embed_pool = pl.kernel(
    _embed_pool_body,
    out_type=jax.ShapeDtypeStruct((B, D), jnp.float32),
    mesh=_mesh,
    scratch_types=_scratch,
)


def kernel(actions, embed_table):
    if actions.dtype != jnp.int32:
        actions = actions.astype(jnp.int32)
    return _embed_pool(actions, embed_table)
